# baseline (speedup 1.0000x reference)
import jax
import jax.numpy as jnp
from jax import lax
from jax.experimental import pallas as pl
from jax.experimental.pallas import tpu as pltpu
from jax.experimental.pallas import tpu_sc as plsc

N, E, D, H, NHID, NOUT, G = 10000, 320000, 128, 128, 256, 128, 64
NC, NS = 2, 16
NW = NC * NS
CH = 128
K = 80
B = 4
R = K // B
EPT = K * CH
EP = NW * EPT
NP = 10112
RPT = NP // NS
f32 = jnp.float32

_HIGH = lax.Precision.HIGHEST


def _mesh():
    return plsc.VectorSubcoreMesh(core_axis_name="c", subcore_axis_name="s")


def _deg_body(dst_hbm, ones_hbm, zn_hbm, out_hbm, dstv, ones, acc):
    cid = lax.axis_index("c")
    sid = lax.axis_index("s")
    wid = cid * NS + sid
    pltpu.sync_copy(zn_hbm.at[pl.ds(sid * RPT, RPT)],
                    acc.at[pl.ds(sid * RPT, RPT)])
    pltpu.sync_copy(ones_hbm, ones)
    pltpu.sync_copy(dst_hbm.at[wid], dstv)
    plsc.subcore_barrier()

    @pl.loop(0, K)
    def _(j):
        pltpu.sync_copy(ones, acc.at[dstv.at[j]], add=True)

    plsc.subcore_barrier()
    pltpu.sync_copy(acc.at[pl.ds(sid * RPT, RPT)],
                    out_hbm.at[cid, pl.ds(sid * RPT, RPT)])


def _deg_call(dst_r, ones128, zn):
    return pl.kernel(
        _deg_body,
        out_type=jax.ShapeDtypeStruct((NC, NP, H), f32),
        mesh=_mesh(),
        scratch_types=[
            pltpu.VMEM((K, CH), jnp.int32),
            pltpu.VMEM((CH, H), f32),
            pltpu.VMEM_SHARED((NP, H), f32),
        ],
    )(dst_r, ones128, zn)


def _scat_body(hp_hbm, src_hbm, dst_hbm, zn_hbm, out_hbm, srcv, dstv, rows,
               acc):
    cid = lax.axis_index("c")
    sid = lax.axis_index("s")
    wid = cid * NS + sid
    pltpu.sync_copy(zn_hbm.at[pl.ds(sid * RPT, RPT)],
                    acc.at[pl.ds(sid * RPT, RPT)])
    pltpu.sync_copy(src_hbm.at[wid], srcv)
    pltpu.sync_copy(dst_hbm.at[wid], dstv)
    plsc.subcore_barrier()

    @pl.loop(0, K)
    def _(j):
        pltpu.sync_copy(hp_hbm.at[srcv.at[j]], rows)
        pltpu.sync_copy(rows, acc.at[dstv.at[j]], add=True)

    plsc.subcore_barrier()
    pltpu.sync_copy(acc.at[pl.ds(sid * RPT, RPT)],
                    out_hbm.at[cid, pl.ds(sid * RPT, RPT)])


def _scat_call(hp, src_r, dst_r, zn):
    return pl.kernel(
        _scat_body,
        out_type=jax.ShapeDtypeStruct((NC, NP, H), f32),
        mesh=_mesh(),
        scratch_types=[
            pltpu.VMEM((K, CH), jnp.int32),
            pltpu.VMEM((K, CH), jnp.int32),
            pltpu.VMEM((CH, H), f32),
            pltpu.VMEM_SHARED((NP, H), f32),
        ],
    )(hp, src_r, dst_r, zn)


def _tc1_body(deg_ref, x_ref, w1_ref, dinv_ref, hp_ref):
    d = deg_ref[0, :, 0:1] + deg_ref[1, :, 0:1] + 1.0
    dinv = lax.rsqrt(d)
    dinv_ref[...] = dinv
    hp_ref[...] = dinv * jnp.dot(x_ref[...], w1_ref[...],
                                 preferred_element_type=f32, precision=_HIGH)


def _tc1(deg, xp, W1):
    return pl.pallas_call(
        _tc1_body,
        out_shape=(jax.ShapeDtypeStruct((NP, 1), f32),
                   jax.ShapeDtypeStruct((NP, H), f32)),
    )(deg, xp, W1)


def _tc_mid_body(s_ref, hp_ref, dinv_ref, b_ref, w_ref, out_ref):
    h = dinv_ref[...] * (s_ref[0] + s_ref[1] + hp_ref[...]) + b_ref[...]
    a = jnp.maximum(h, 0.0)
    out_ref[...] = dinv_ref[...] * jnp.dot(a, w_ref[...],
                                           preferred_element_type=f32,
                                           precision=_HIGH)


def _tc_mid(s, hp, dinv, b, Wn):
    return pl.pallas_call(
        _tc_mid_body,
        out_shape=jax.ShapeDtypeStruct((NP, H), f32),
    )(s, hp, dinv, b, Wn)


def _tc_fin_body(s_ref, hp_ref, dinv_ref, b3_ref, batch_ref,
                 wh1_ref, bh1_ref, wh2_ref, bh2_ref, z_ref):
    h3 = dinv_ref[...] * (s_ref[0] + s_ref[1] + hp_ref[...]) + b3_ref[...]
    gid = lax.broadcasted_iota(jnp.int32, (G, NP), 0)
    mask = (batch_ref[...] == gid).astype(f32)
    sums = jnp.dot(mask, h3, preferred_element_type=f32, precision=_HIGH)
    counts = jnp.sum(mask, axis=1, keepdims=True)
    p = sums / jnp.maximum(counts, 1.0)
    a = jnp.maximum(jnp.dot(p, wh1_ref[...], preferred_element_type=f32,
                            precision=_HIGH) + bh1_ref[...], 0.0)
    z_ref[...] = jnp.dot(a, wh2_ref[...], preferred_element_type=f32,
                         precision=_HIGH) + bh2_ref[...]


def _tc_fin(s, hp, dinv, b3, batch2d, Wh1, bh1, Wh2, bh2):
    return pl.pallas_call(
        _tc_fin_body,
        out_shape=jax.ShapeDtypeStruct((G, NOUT), f32),
    )(s, hp, dinv, b3, batch2d, Wh1, bh1, Wh2, bh2)


def kernel(x, edge_index, batch, W1, b1, W2, b2, W3, b3, Wh1, bh1, Wh2, bh2):
    src = edge_index[0]
    dst = edge_index[1]
    pad = EP - E
    src_r = jnp.concatenate(
        [src, jnp.zeros((pad,), jnp.int32)]).reshape(NW, K, CH)
    dst_r = jnp.concatenate(
        [dst, N + (jnp.arange(pad, dtype=jnp.int32) % 16)]).reshape(NW, K, CH)
    xp = jnp.pad(x, ((0, NP - N), (0, 0)))
    batch2d = jnp.pad(batch, (0, NP - N), constant_values=G).reshape(1, NP)
    ones128 = jnp.ones((CH, H), f32)
    zn = jnp.zeros((NP, H), f32)
    b1r, b2r, b3r = b1.reshape(1, H), b2.reshape(1, H), b3.reshape(1, H)
    bh1r, bh2r = bh1.reshape(1, NHID), bh2.reshape(1, NOUT)

    deg = _deg_call(dst_r, ones128, zn)
    dinv, h1p = _tc1(deg, xp, W1)
    s1 = _scat_call(h1p, src_r, dst_r, zn)
    h2p = _tc_mid(s1, h1p, dinv, b1r, W2)
    s2 = _scat_call(h2p, src_r, dst_r, zn)
    h3p = _tc_mid(s2, h2p, dinv, b2r, W3)
    s3 = _scat_call(h3p, src_r, dst_r, zn)
    return _tc_fin(s3, h3p, dinv, b3r, batch2d, Wh1, bh1r, Wh2, bh2r)


# speedup vs baseline: 2.7732x; 2.7732x over previous
import jax
import jax.numpy as jnp
from jax import lax
from jax.experimental import pallas as pl
from jax.experimental.pallas import tpu as pltpu
from jax.experimental.pallas import tpu_sc as plsc

N, E, D, H, NHID, NOUT, G = 10000, 320000, 128, 128, 256, 128, 64
NC, NS = 2, 16
NW = NC * NS
CH = 128
K = 79
EPT = K * CH
EP = NW * EPT
NP = 10112
RPT = NP // NS
f32 = jnp.float32

_HIGH = lax.Precision.HIGHEST


def _mesh():
    return plsc.VectorSubcoreMesh(core_axis_name="c", subcore_axis_name="s")


def _deg_body(dst_hbm, ones_hbm, zn_hbm, out_hbm, dstv, ones, acc):
    cid = lax.axis_index("c")
    sid = lax.axis_index("s")
    wid = cid * NS + sid
    pltpu.sync_copy(zn_hbm.at[pl.ds(sid * RPT, RPT)],
                    acc.at[pl.ds(sid * RPT, RPT)])
    pltpu.sync_copy(ones_hbm, ones)
    pltpu.sync_copy(dst_hbm.at[wid], dstv)
    plsc.subcore_barrier()

    @pl.loop(0, K)
    def _(j):
        pltpu.sync_copy(ones, acc.at[dstv.at[j]], add=True)

    plsc.subcore_barrier()
    pltpu.sync_copy(acc.at[pl.ds(sid * RPT, RPT)],
                    out_hbm.at[cid, pl.ds(sid * RPT, RPT)])


def _deg_call(dst_r, ones128, zn):
    return pl.kernel(
        _deg_body,
        out_type=jax.ShapeDtypeStruct((NC, NP, H), f32),
        mesh=_mesh(),
        scratch_types=[
            pltpu.VMEM((K, CH), jnp.int32),
            pltpu.VMEM((CH, H), f32),
            pltpu.VMEM_SHARED((NP, H), f32),
        ],
    )(dst_r, ones128, zn)


def _scat_body(hp_hbm, src_hbm, dst_hbm, zn_hbm, out_hbm, srcv, dstv, rows,
               acc):
    cid = lax.axis_index("c")
    sid = lax.axis_index("s")
    wid = cid * NS + sid
    pltpu.sync_copy(zn_hbm.at[pl.ds(sid * RPT, RPT)],
                    acc.at[pl.ds(sid * RPT, RPT)])
    pltpu.sync_copy(src_hbm.at[wid], srcv)
    pltpu.sync_copy(dst_hbm.at[wid], dstv)
    plsc.subcore_barrier()

    @pl.loop(0, K)
    def _(j):
        pltpu.sync_copy(hp_hbm.at[srcv.at[j]], rows)
        pltpu.sync_copy(rows, acc.at[dstv.at[j]], add=True)

    plsc.subcore_barrier()
    pltpu.sync_copy(acc.at[pl.ds(sid * RPT, RPT)],
                    out_hbm.at[cid, pl.ds(sid * RPT, RPT)])


def _scat_call(hp, src_r, dst_r, zn):
    return pl.kernel(
        _scat_body,
        out_type=jax.ShapeDtypeStruct((NC, NP, H), f32),
        mesh=_mesh(),
        scratch_types=[
            pltpu.VMEM((K, CH), jnp.int32),
            pltpu.VMEM((K, CH), jnp.int32),
            pltpu.VMEM((CH, H), f32),
            pltpu.VMEM_SHARED((NP, H), f32),
        ],
    )(hp, src_r, dst_r, zn)


def _tc1_body(deg_ref, x_ref, w1_ref, dinv_ref, hp_ref):
    d = deg_ref[0, :, 0:1] + deg_ref[1, :, 0:1] + 1.0
    dinv = lax.rsqrt(d)
    dinv_ref[...] = dinv
    hp_ref[...] = dinv * jnp.dot(x_ref[...], w1_ref[...],
                                 preferred_element_type=f32, precision=_HIGH)


def _tc1(deg, xp, W1):
    return pl.pallas_call(
        _tc1_body,
        out_shape=(jax.ShapeDtypeStruct((NP, 1), f32),
                   jax.ShapeDtypeStruct((NP, H), f32)),
    )(deg, xp, W1)


def _tc_mid_body(s_ref, hp_ref, dinv_ref, b_ref, w_ref, out_ref):
    h = dinv_ref[...] * (s_ref[0] + s_ref[1] + hp_ref[...]) + b_ref[...]
    a = jnp.maximum(h, 0.0)
    out_ref[...] = dinv_ref[...] * jnp.dot(a, w_ref[...],
                                           preferred_element_type=f32,
                                           precision=_HIGH)


def _tc_mid(s, hp, dinv, b, Wn):
    return pl.pallas_call(
        _tc_mid_body,
        out_shape=jax.ShapeDtypeStruct((NP, H), f32),
    )(s, hp, dinv, b, Wn)


def _tc_fin_body(s_ref, hp_ref, dinv_ref, b3_ref, batch_ref,
                 wh1_ref, bh1_ref, wh2_ref, bh2_ref, z_ref):
    h3 = dinv_ref[...] * (s_ref[0] + s_ref[1] + hp_ref[...]) + b3_ref[...]
    gid = lax.broadcasted_iota(jnp.int32, (G, NP), 0)
    mask = (batch_ref[...] == gid).astype(f32)
    sums = jnp.dot(mask, h3, preferred_element_type=f32, precision=_HIGH)
    counts = jnp.sum(mask, axis=1, keepdims=True)
    p = sums / jnp.maximum(counts, 1.0)
    a = jnp.maximum(jnp.dot(p, wh1_ref[...], preferred_element_type=f32,
                            precision=_HIGH) + bh1_ref[...], 0.0)
    z_ref[...] = jnp.dot(a, wh2_ref[...], preferred_element_type=f32,
                         precision=_HIGH) + bh2_ref[...]


def _tc_fin(s, hp, dinv, b3, batch2d, Wh1, bh1, Wh2, bh2):
    return pl.pallas_call(
        _tc_fin_body,
        out_shape=jax.ShapeDtypeStruct((G, NOUT), f32),
    )(s, hp, dinv, b3, batch2d, Wh1, bh1, Wh2, bh2)


def kernel(x, edge_index, batch, W1, b1, W2, b2, W3, b3, Wh1, bh1, Wh2, bh2):
    src = edge_index[0]
    dst = edge_index[1]
    pad = EP - E
    ar = jnp.arange(pad, dtype=jnp.int32)
    src_r = jnp.concatenate([src, ar * 37 % N]).reshape(NW, K, CH)
    dst_r = jnp.concatenate([dst, N + ar % (NP - N)]).reshape(NW, K, CH)
    xp = jnp.pad(x, ((0, NP - N), (0, 0)))
    batch2d = jnp.pad(batch, (0, NP - N), constant_values=G).reshape(1, NP)
    ones128 = jnp.ones((CH, H), f32)
    zn = jnp.zeros((NP, H), f32)
    b1r, b2r, b3r = b1.reshape(1, H), b2.reshape(1, H), b3.reshape(1, H)
    bh1r, bh2r = bh1.reshape(1, NHID), bh2.reshape(1, NOUT)

    deg = _deg_call(dst_r, ones128, zn)
    dinv, h1p = _tc1(deg, xp, W1)
    s1 = _scat_call(h1p, src_r, dst_r, zn)
    h2p = _tc_mid(s1, h1p, dinv, b1r, W2)
    s2 = _scat_call(h2p, src_r, dst_r, zn)
    h3p = _tc_mid(s2, h2p, dinv, b2r, W3)
    s3 = _scat_call(h3p, src_r, dst_r, zn)
    return _tc_fin(s3, h3p, dinv, b3r, batch2d, Wh1, bh1r, Wh2, bh2r)


# speedup vs baseline: 2.8754x; 1.0369x over previous
import jax
import jax.numpy as jnp
from jax import lax
from jax.experimental import pallas as pl
from jax.experimental.pallas import tpu as pltpu
from jax.experimental.pallas import tpu_sc as plsc

N, E, D, H, NHID, NOUT, G = 10000, 320000, 128, 128, 256, 128, 64
NC, NS = 2, 16
NW = NC * NS
CH = 128
K = 79
EPT = K * CH
EP = NW * EPT
NP = 10112
RPT = NP // NS
f32 = jnp.float32

_HIGH = lax.Precision.HIGHEST


def _mesh():
    return plsc.VectorSubcoreMesh(core_axis_name="c", subcore_axis_name="s")


def _deg_body(dst_hbm, ones_hbm, zn_hbm, out_hbm, dstv, ones, acc):
    cid = lax.axis_index("c")
    sid = lax.axis_index("s")
    wid = cid * NS + sid
    pltpu.sync_copy(zn_hbm.at[pl.ds(sid * RPT, RPT)],
                    acc.at[pl.ds(sid * RPT, RPT)])
    pltpu.sync_copy(ones_hbm, ones)
    pltpu.sync_copy(dst_hbm.at[wid], dstv)
    plsc.subcore_barrier()

    @pl.loop(0, K)
    def _(j):
        pltpu.sync_copy(ones, acc.at[dstv.at[j]], add=True)

    plsc.subcore_barrier()
    pltpu.sync_copy(acc.at[pl.ds(sid * RPT, RPT)],
                    out_hbm.at[cid, pl.ds(sid * RPT, RPT)])


def _deg_call(dst_r, ones128, zn):
    return pl.kernel(
        _deg_body,
        out_type=jax.ShapeDtypeStruct((NC, NP, H), f32),
        mesh=_mesh(),
        scratch_types=[
            pltpu.VMEM((K, CH), jnp.int32),
            pltpu.VMEM((CH, H), f32),
            pltpu.VMEM_SHARED((NP, H), f32),
        ],
    )(dst_r, ones128, zn)


def _scat_body(hp_hbm, src_hbm, dst_hbm, zn_hbm, out_hbm, srcv, dstv, rows,
               acc, gs0, gs1):
    cid = lax.axis_index("c")
    sid = lax.axis_index("s")
    wid = cid * NS + sid
    pltpu.sync_copy(zn_hbm.at[pl.ds(sid * RPT, RPT)],
                    acc.at[pl.ds(sid * RPT, RPT)])
    pltpu.sync_copy(src_hbm.at[wid], srcv)
    pltpu.sync_copy(dst_hbm.at[wid], dstv)
    plsc.subcore_barrier()

    @pl.loop(0, K)
    def _(j):
        c0 = pltpu.async_copy(hp_hbm.at[srcv.at[j, pl.ds(0, CH // 2)]],
                              rows.at[pl.ds(0, CH // 2)], gs0)
        c1 = pltpu.async_copy(hp_hbm.at[srcv.at[j, pl.ds(CH // 2, CH // 2)]],
                              rows.at[pl.ds(CH // 2, CH // 2)], gs1)
        c0.wait()
        c1.wait()
        pltpu.sync_copy(rows, acc.at[dstv.at[j]], add=True)

    plsc.subcore_barrier()
    pltpu.sync_copy(acc.at[pl.ds(sid * RPT, RPT)],
                    out_hbm.at[cid, pl.ds(sid * RPT, RPT)])


def _scat_call(hp, src_r, dst_r, zn):
    return pl.kernel(
        _scat_body,
        out_type=jax.ShapeDtypeStruct((NC, NP, H), f32),
        mesh=_mesh(),
        scratch_types=[
            pltpu.VMEM((K, CH), jnp.int32),
            pltpu.VMEM((K, CH), jnp.int32),
            pltpu.VMEM((CH, H), f32),
            pltpu.VMEM_SHARED((NP, H), f32),
            pltpu.SemaphoreType.DMA,
            pltpu.SemaphoreType.DMA,
        ],
    )(hp, src_r, dst_r, zn)


def _tc1_body(deg_ref, x_ref, w1_ref, dinv_ref, hp_ref):
    d = deg_ref[0, :, 0:1] + deg_ref[1, :, 0:1] + 1.0
    dinv = lax.rsqrt(d)
    dinv_ref[...] = dinv
    hp_ref[...] = dinv * jnp.dot(x_ref[...], w1_ref[...],
                                 preferred_element_type=f32, precision=_HIGH)


def _tc1(deg, xp, W1):
    return pl.pallas_call(
        _tc1_body,
        out_shape=(jax.ShapeDtypeStruct((NP, 1), f32),
                   jax.ShapeDtypeStruct((NP, H), f32)),
    )(deg, xp, W1)


def _tc_mid_body(s_ref, hp_ref, dinv_ref, b_ref, w_ref, out_ref):
    h = dinv_ref[...] * (s_ref[0] + s_ref[1] + hp_ref[...]) + b_ref[...]
    a = jnp.maximum(h, 0.0)
    out_ref[...] = dinv_ref[...] * jnp.dot(a, w_ref[...],
                                           preferred_element_type=f32,
                                           precision=_HIGH)


def _tc_mid(s, hp, dinv, b, Wn):
    return pl.pallas_call(
        _tc_mid_body,
        out_shape=jax.ShapeDtypeStruct((NP, H), f32),
    )(s, hp, dinv, b, Wn)


def _tc_fin_body(s_ref, hp_ref, dinv_ref, b3_ref, batch_ref,
                 wh1_ref, bh1_ref, wh2_ref, bh2_ref, z_ref):
    h3 = dinv_ref[...] * (s_ref[0] + s_ref[1] + hp_ref[...]) + b3_ref[...]
    gid = lax.broadcasted_iota(jnp.int32, (G, NP), 0)
    mask = (batch_ref[...] == gid).astype(f32)
    sums = jnp.dot(mask, h3, preferred_element_type=f32, precision=_HIGH)
    counts = jnp.sum(mask, axis=1, keepdims=True)
    p = sums / jnp.maximum(counts, 1.0)
    a = jnp.maximum(jnp.dot(p, wh1_ref[...], preferred_element_type=f32,
                            precision=_HIGH) + bh1_ref[...], 0.0)
    z_ref[...] = jnp.dot(a, wh2_ref[...], preferred_element_type=f32,
                         precision=_HIGH) + bh2_ref[...]


def _tc_fin(s, hp, dinv, b3, batch2d, Wh1, bh1, Wh2, bh2):
    return pl.pallas_call(
        _tc_fin_body,
        out_shape=jax.ShapeDtypeStruct((G, NOUT), f32),
    )(s, hp, dinv, b3, batch2d, Wh1, bh1, Wh2, bh2)


def kernel(x, edge_index, batch, W1, b1, W2, b2, W3, b3, Wh1, bh1, Wh2, bh2):
    src = edge_index[0]
    dst = edge_index[1]
    pad = EP - E
    ar = jnp.arange(pad, dtype=jnp.int32)
    src_r = jnp.concatenate([src, ar * 37 % N]).reshape(NW, K, CH)
    dst_r = jnp.concatenate([dst, N + ar % (NP - N)]).reshape(NW, K, CH)
    xp = jnp.pad(x, ((0, NP - N), (0, 0)))
    batch2d = jnp.pad(batch, (0, NP - N), constant_values=G).reshape(1, NP)
    ones128 = jnp.ones((CH, H), f32)
    zn = jnp.zeros((NP, H), f32)
    b1r, b2r, b3r = b1.reshape(1, H), b2.reshape(1, H), b3.reshape(1, H)
    bh1r, bh2r = bh1.reshape(1, NHID), bh2.reshape(1, NOUT)

    deg = _deg_call(dst_r, ones128, zn)
    dinv, h1p = _tc1(deg, xp, W1)
    s1 = _scat_call(h1p, src_r, dst_r, zn)
    h2p = _tc_mid(s1, h1p, dinv, b1r, W2)
    s2 = _scat_call(h2p, src_r, dst_r, zn)
    h3p = _tc_mid(s2, h2p, dinv, b2r, W3)
    s3 = _scat_call(h3p, src_r, dst_r, zn)
    return _tc_fin(s3, h3p, dinv, b3r, batch2d, Wh1, bh1r, Wh2, bh2r)


# speedup vs baseline: 2.8770x; 1.0006x over previous
import jax
import jax.numpy as jnp
from jax import lax
from jax.experimental import pallas as pl
from jax.experimental.pallas import tpu as pltpu
from jax.experimental.pallas import tpu_sc as plsc

N, E, D, H, NHID, NOUT, G = 10000, 320000, 128, 128, 256, 128, 64
NC, NS = 2, 16
NW = NC * NS
CH = 128
K = 79
EPT = K * CH
EP = NW * EPT
NP = 10112
RPT = NP // NS
f32 = jnp.float32

_HIGH = lax.Precision.HIGHEST


def _mesh():
    return plsc.VectorSubcoreMesh(core_axis_name="c", subcore_axis_name="s")


def _deg_body(dst_hbm, ones_hbm, zn_hbm, out_hbm, dstv, ones, acc, dsem):
    cid = lax.axis_index("c")
    sid = lax.axis_index("s")
    wid = cid * NS + sid
    pltpu.sync_copy(zn_hbm.at[pl.ds(sid * RPT, RPT)],
                    acc.at[pl.ds(sid * RPT, RPT)])
    pltpu.sync_copy(ones_hbm, ones)
    pltpu.sync_copy(dst_hbm.at[wid], dstv)
    plsc.subcore_barrier()

    @pl.loop(0, K - K % 8, step=8)
    def _(j0):
        for b in range(8):
            pltpu.async_copy(ones, acc.at[dstv.at[j0 + b]], dsem, add=True)
        for b in range(8):
            pltpu.make_async_copy(ones, acc.at[dstv.at[j0 + b]], dsem).wait()

    for j in range(K - K % 8, K):
        pltpu.async_copy(ones, acc.at[dstv.at[j]], dsem, add=True)
    for j in range(K - K % 8, K):
        pltpu.make_async_copy(ones, acc.at[dstv.at[j]], dsem).wait()

    plsc.subcore_barrier()
    pltpu.sync_copy(acc.at[pl.ds(sid * RPT, RPT)],
                    out_hbm.at[cid, pl.ds(sid * RPT, RPT)])


def _deg_call(dst_r, ones128, zn):
    return pl.kernel(
        _deg_body,
        out_type=jax.ShapeDtypeStruct((NC, NP, H), f32),
        mesh=_mesh(),
        scratch_types=[
            pltpu.VMEM((K, CH), jnp.int32),
            pltpu.VMEM((CH, H), f32),
            pltpu.VMEM_SHARED((NP, H), f32),
            pltpu.SemaphoreType.DMA,
        ],
    )(dst_r, ones128, zn)


def _scat_body(hp_hbm, src_hbm, dst_hbm, zn_hbm, out_hbm, srcv, dstv, rows,
               acc, gs0, gs1):
    cid = lax.axis_index("c")
    sid = lax.axis_index("s")
    wid = cid * NS + sid
    pltpu.sync_copy(zn_hbm.at[pl.ds(sid * RPT, RPT)],
                    acc.at[pl.ds(sid * RPT, RPT)])
    pltpu.sync_copy(src_hbm.at[wid], srcv)
    pltpu.sync_copy(dst_hbm.at[wid], dstv)
    plsc.subcore_barrier()

    @pl.loop(0, K)
    def _(j):
        c0 = pltpu.async_copy(hp_hbm.at[srcv.at[j, pl.ds(0, CH // 2)]],
                              rows.at[pl.ds(0, CH // 2)], gs0)
        c1 = pltpu.async_copy(hp_hbm.at[srcv.at[j, pl.ds(CH // 2, CH // 2)]],
                              rows.at[pl.ds(CH // 2, CH // 2)], gs1)
        c0.wait()
        c1.wait()
        pltpu.sync_copy(rows, acc.at[dstv.at[j]], add=True)

    plsc.subcore_barrier()
    pltpu.sync_copy(acc.at[pl.ds(sid * RPT, RPT)],
                    out_hbm.at[cid, pl.ds(sid * RPT, RPT)])


def _scat_call(hp, src_r, dst_r, zn):
    return pl.kernel(
        _scat_body,
        out_type=jax.ShapeDtypeStruct((NC, NP, H), f32),
        mesh=_mesh(),
        scratch_types=[
            pltpu.VMEM((K, CH), jnp.int32),
            pltpu.VMEM((K, CH), jnp.int32),
            pltpu.VMEM((CH, H), f32),
            pltpu.VMEM_SHARED((NP, H), f32),
            pltpu.SemaphoreType.DMA,
            pltpu.SemaphoreType.DMA,
        ],
    )(hp, src_r, dst_r, zn)


def _tc1_body(deg_ref, x_ref, w1_ref, dinv_ref, hp_ref):
    d = deg_ref[0, :, 0:1] + deg_ref[1, :, 0:1] + 1.0
    dinv = lax.rsqrt(d)
    dinv_ref[...] = dinv
    hp_ref[...] = dinv * jnp.dot(x_ref[...], w1_ref[...],
                                 preferred_element_type=f32, precision=_HIGH)


def _tc1(deg, xp, W1):
    return pl.pallas_call(
        _tc1_body,
        out_shape=(jax.ShapeDtypeStruct((NP, 1), f32),
                   jax.ShapeDtypeStruct((NP, H), f32)),
    )(deg, xp, W1)


def _tc_mid_body(s_ref, hp_ref, dinv_ref, b_ref, w_ref, out_ref):
    h = dinv_ref[...] * (s_ref[0] + s_ref[1] + hp_ref[...]) + b_ref[...]
    a = jnp.maximum(h, 0.0)
    out_ref[...] = dinv_ref[...] * jnp.dot(a, w_ref[...],
                                           preferred_element_type=f32,
                                           precision=_HIGH)


def _tc_mid(s, hp, dinv, b, Wn):
    return pl.pallas_call(
        _tc_mid_body,
        out_shape=jax.ShapeDtypeStruct((NP, H), f32),
    )(s, hp, dinv, b, Wn)


def _tc_fin_body(s_ref, hp_ref, dinv_ref, b3_ref, batch_ref,
                 wh1_ref, bh1_ref, wh2_ref, bh2_ref, z_ref):
    h3 = dinv_ref[...] * (s_ref[0] + s_ref[1] + hp_ref[...]) + b3_ref[...]
    gid = lax.broadcasted_iota(jnp.int32, (G, NP), 0)
    mask = (batch_ref[...] == gid).astype(f32)
    sums = jnp.dot(mask, h3, preferred_element_type=f32, precision=_HIGH)
    counts = jnp.sum(mask, axis=1, keepdims=True)
    p = sums / jnp.maximum(counts, 1.0)
    a = jnp.maximum(jnp.dot(p, wh1_ref[...], preferred_element_type=f32,
                            precision=_HIGH) + bh1_ref[...], 0.0)
    z_ref[...] = jnp.dot(a, wh2_ref[...], preferred_element_type=f32,
                         precision=_HIGH) + bh2_ref[...]


def _tc_fin(s, hp, dinv, b3, batch2d, Wh1, bh1, Wh2, bh2):
    return pl.pallas_call(
        _tc_fin_body,
        out_shape=jax.ShapeDtypeStruct((G, NOUT), f32),
    )(s, hp, dinv, b3, batch2d, Wh1, bh1, Wh2, bh2)


def kernel(x, edge_index, batch, W1, b1, W2, b2, W3, b3, Wh1, bh1, Wh2, bh2):
    src = edge_index[0]
    dst = edge_index[1]
    pad = EP - E
    ar = jnp.arange(pad, dtype=jnp.int32)
    src_r = jnp.concatenate([src, ar * 37 % N]).reshape(NW, K, CH)
    dst_r = jnp.concatenate([dst, N + ar % (NP - N)]).reshape(NW, K, CH)
    xp = jnp.pad(x, ((0, NP - N), (0, 0)))
    batch2d = jnp.pad(batch, (0, NP - N), constant_values=G).reshape(1, NP)
    ones128 = jnp.ones((CH, H), f32)
    zn = jnp.zeros((NP, H), f32)
    b1r, b2r, b3r = b1.reshape(1, H), b2.reshape(1, H), b3.reshape(1, H)
    bh1r, bh2r = bh1.reshape(1, NHID), bh2.reshape(1, NOUT)

    deg = _deg_call(dst_r, ones128, zn)
    dinv, h1p = _tc1(deg, xp, W1)
    s1 = _scat_call(h1p, src_r, dst_r, zn)
    h2p = _tc_mid(s1, h1p, dinv, b1r, W2)
    s2 = _scat_call(h2p, src_r, dst_r, zn)
    h3p = _tc_mid(s2, h2p, dinv, b2r, W3)
    s3 = _scat_call(h3p, src_r, dst_r, zn)
    return _tc_fin(s3, h3p, dinv, b3r, batch2d, Wh1, bh1r, Wh2, bh2r)
